# baseline (speedup 1.0000x reference)
import functools

import jax
import jax.numpy as jnp
from jax import lax
from jax.experimental import pallas as pl
from jax.experimental.pallas import tpu as pltpu
from jax.experimental.pallas import tpu_sc as plsc

_B = 2
_S = 2048
_H = 1024
_NH = 16
_HD = _H // _NH
_I = 4 * _H
_R = 3
_ALPHA = 0.1
_CAP = 0.5
_K = max(1, int(_CAP * _S))
_M = _B * _K
_RB = _R * _B
_ITILE = 512
_IT = _I // _ITILE


def _dotT(a, b, prec=None):
    return lax.dot_general(a, b, (((1,), (1,)), ((), ())),
                           preferred_element_type=jnp.float32,
                           precision=prec)


def _router_body(h_ref, wr_ref, out_ref):
    out_ref[...] = lax.dot_general(
        h_ref[...], wr_ref[...], (((1,), (0,)), ((), ())),
        preferred_element_type=jnp.float32,
        precision=lax.Precision.HIGHEST)


def _router(hidden_flat, wr_pad):
    n = hidden_flat.shape[0]
    tm = 1024
    return pl.pallas_call(
        _router_body,
        grid=(n // tm,),
        in_specs=[
            pl.BlockSpec((tm, _H), lambda i: (i, 0)),
            pl.BlockSpec((_H, 128), lambda i: (0, 0)),
        ],
        out_specs=pl.BlockSpec((tm, 128), lambda i: (i, 0)),
        out_shape=jax.ShapeDtypeStruct((n, 128), jnp.float32),
    )(hidden_flat, wr_pad)


def _sc_gather(table, idx_flat, n_rows):
    info = plsc.get_sparse_core_info()
    nw = info.num_cores * info.num_subcores
    rpw = n_rows // nw
    ch = 64
    mesh = plsc.VectorSubcoreMesh(core_axis_name="c", subcore_axis_name="s")

    @functools.partial(
        pl.kernel, mesh=mesh,
        out_type=jax.ShapeDtypeStruct((n_rows, _H), jnp.float32),
        scratch_types=[
            pltpu.VMEM((ch,), jnp.int32),
            pltpu.VMEM((ch, _H), jnp.float32),
            pltpu.SemaphoreType.DMA,
        ],
    )
    def k(table_hbm, idx_hbm, out_hbm, idx_v, rows_v, sem):
        wid = lax.axis_index("s") * info.num_cores + lax.axis_index("c")
        base = wid * rpw
        for c in range(rpw // ch):
            off = base + c * ch
            pltpu.sync_copy(idx_hbm.at[pl.ds(off, ch)], idx_v)
            pltpu.async_copy(table_hbm.at[idx_v], rows_v, sem).wait()
            pltpu.sync_copy(rows_v, out_hbm.at[pl.ds(off, ch)])

    return k(table, idx_flat)


def _ln(x, w, b):
    mu = jnp.mean(x, axis=-1, keepdims=True)
    xc = x - mu
    var = jnp.mean(xc * xc, axis=-1, keepdims=True)
    return xc * lax.rsqrt(var + 1e-6) * w + b


_MT = 512


def _qkv_body(sel_ref, wq_ref, wk_ref, wv_ref, l1w_ref, l1b_ref,
              q_ref, k_ref, v_ref):
    hn = _ln(sel_ref[...], l1w_ref[...], l1b_ref[...]).astype(jnp.bfloat16)
    q_ref[...] = _dotT(hn, wq_ref[...].astype(jnp.bfloat16)).astype(jnp.bfloat16)
    k_ref[...] = _dotT(hn, wk_ref[...].astype(jnp.bfloat16)).astype(jnp.bfloat16)
    v_ref[...] = _dotT(hn, wv_ref[...].astype(jnp.bfloat16)).astype(jnp.bfloat16)


def _qkv(sel, Wq, Wk, Wv, l1w, l1b):
    spec_w = pl.BlockSpec((None, _H, _H), lambda r, m: (r, 0, 0))
    spec_o = pl.BlockSpec((None, _MT, _H), lambda r, m: (r, m, 0))
    spec_l = pl.BlockSpec((None, 1, _H), lambda r, m: (r, 0, 0))
    return pl.pallas_call(
        _qkv_body,
        grid=(_R, _M // _MT),
        in_specs=[
            pl.BlockSpec((None, _MT, _H), lambda r, m: (r, m, 0)),
            spec_w, spec_w, spec_w, spec_l, spec_l,
        ],
        out_specs=[spec_o, spec_o, spec_o],
        out_shape=[jax.ShapeDtypeStruct((_R, _M, _H), jnp.bfloat16)] * 3,
    )(sel, Wq, Wk, Wv, l1w, l1b)


def _attn_body(q_ref, k_ref, v_ref, o_ref):
    s = lax.dot_general(q_ref[...], k_ref[...], (((1,), (1,)), ((), ())),
                        preferred_element_type=jnp.float32)
    s = s * (1.0 / (_HD ** 0.5))
    m = jnp.max(s, axis=-1, keepdims=True)
    e = jnp.exp(s - m)
    p = (e / jnp.sum(e, axis=-1, keepdims=True)).astype(jnp.bfloat16)
    o_ref[...] = lax.dot_general(p, v_ref[...], (((1,), (0,)), ((), ())),
                                 preferred_element_type=jnp.float32
                                 ).astype(jnp.bfloat16)


def _attention(qh, kh, vh):
    g = _RB * _NH
    spec = pl.BlockSpec((None, _K, _HD), lambda i: (i, 0, 0))
    return pl.pallas_call(
        _attn_body,
        grid=(g,),
        in_specs=[spec, spec, spec],
        out_specs=spec,
        out_shape=jax.ShapeDtypeStruct((g, _K, _HD), jnp.bfloat16),
    )(qh, kh, vh)


def _oproj_body(sel_ref, attn_ref, wo_ref, l2w_ref, l2b_ref, h1_ref, hn2_ref):
    h1 = sel_ref[...] + _dotT(attn_ref[...], wo_ref[...].astype(jnp.bfloat16))
    h1_ref[...] = h1
    hn2_ref[...] = _ln(h1, l2w_ref[...], l2b_ref[...]).astype(jnp.bfloat16)


def _oproj(sel, attn, Wo, l2w, l2b):
    spec_m = pl.BlockSpec((None, _MT, _H), lambda r, m: (r, m, 0))
    spec_l = pl.BlockSpec((None, 1, _H), lambda r, m: (r, 0, 0))
    return pl.pallas_call(
        _oproj_body,
        grid=(_R, _M // _MT),
        in_specs=[
            spec_m, spec_m,
            pl.BlockSpec((None, _H, _H), lambda r, m: (r, 0, 0)),
            spec_l, spec_l,
        ],
        out_specs=[spec_m, spec_m],
        out_shape=[
            jax.ShapeDtypeStruct((_R, _M, _H), jnp.float32),
            jax.ShapeDtypeStruct((_R, _M, _H), jnp.bfloat16),
        ],
    )(sel, attn, Wo, l2w, l2b)


def _mlp_body(hn2_ref, wg_ref, wu_ref, wd_ref, h1_ref, w_ref, out_ref):
    hn2 = hn2_ref[...]
    g = _dotT(hn2, wg_ref[...].astype(jnp.bfloat16))
    g = g * jax.nn.sigmoid(g)
    u = _dotT(hn2, wu_ref[...].astype(jnp.bfloat16))
    prod = (g * u).astype(jnp.bfloat16)
    part = _dotT(prod, wd_ref[...].astype(jnp.bfloat16))
    j = pl.program_id(2)

    @pl.when(j == 0)
    def _():
        out_ref[...] = h1_ref[...] + part

    @pl.when(j > 0)
    def _():
        out_ref[...] += part

    @pl.when(j == _IT - 1)
    def _():
        out_ref[...] *= w_ref[...]


def _mlp(hn2, Wg, Wu, Wd, h1, w):
    mtm = 1024
    return pl.pallas_call(
        _mlp_body,
        grid=(_R, _M // mtm, _IT),
        in_specs=[
            pl.BlockSpec((None, mtm, _H), lambda r, m, j: (r, m, 0)),
            pl.BlockSpec((None, _ITILE, _H), lambda r, m, j: (r, j, 0)),
            pl.BlockSpec((None, _ITILE, _H), lambda r, m, j: (r, j, 0)),
            pl.BlockSpec((None, _H, _ITILE), lambda r, m, j: (r, 0, j)),
            pl.BlockSpec((None, mtm, _H), lambda r, m, j: (r, m, 0)),
            pl.BlockSpec((None, mtm, 1), lambda r, m, j: (r, m, 0)),
        ],
        out_specs=pl.BlockSpec((None, mtm, _H), lambda r, m, j: (r, m, 0)),
        out_shape=jax.ShapeDtypeStruct((_R, _M, _H), jnp.float32),
    )(hn2, Wg, Wu, Wd, h1, w)


def _scatter_body(hid_ref, wtd_ref, idx_ref, out_ref):
    step = pl.program_id(0)

    @pl.when(step == 0)
    def _():
        out_ref[...] = hid_ref[...]

    def body(i, _):
        g = idx_ref[0, i]
        out_ref[pl.ds(g, 1)] += wtd_ref[pl.ds(i, 1)]
        return 0

    lax.fori_loop(0, _K, body, 0)


def _scatter_add(hidden_b, weighted_b, idx_b):
    return pl.pallas_call(
        _scatter_body,
        grid=(_R,),
        in_specs=[
            pl.BlockSpec((_S, 8, 128), lambda i: (0, 0, 0)),
            pl.BlockSpec((None, _K, 8, 128), lambda i: (i, 0, 0, 0)),
            pl.BlockSpec((None, 1, _K), lambda i: (i, 0, 0),
                         memory_space=pltpu.SMEM),
        ],
        out_specs=pl.BlockSpec((_S, 8, 128), lambda i: (0, 0, 0)),
        out_shape=jax.ShapeDtypeStruct((_S, 8, 128), jnp.float32),
    )(hidden_b, weighted_b, idx_b)


def kernel(hidden_states, Wr, Wq, Wk, Wv, Wo, Wg, Wu, Wd,
           ln1w, ln1b, ln2w, ln2b):
    b, s, h = hidden_states.shape
    hid_flat = hidden_states.reshape(b * s, h)

    wr_pad = jnp.zeros((h, 128), jnp.float32).at[:, :_R].set(
        Wr.reshape(_R, h).T)
    logits_all = _router(hid_flat, wr_pad)
    logits = logits_all.reshape(b, s, 128)[:, :, :_R]
    logits = jnp.transpose(logits, (2, 0, 1))
    router_logits = logits[..., None]

    probs = jax.nn.sigmoid(logits) * _ALPHA
    w, idx = lax.top_k(probs, _K)
    order = jnp.argsort(idx, axis=-1)
    idx = jnp.take_along_axis(idx, order, axis=-1)
    w = jnp.take_along_axis(w, order, axis=-1)
    gidx = idx + jnp.arange(b, dtype=idx.dtype)[None, :, None] * s
    gidx_flat = gidx.reshape(-1).astype(jnp.int32)

    sel = _sc_gather(hid_flat, gidx_flat, _RB * _K)
    sel = sel.reshape(_R, _M, h)

    l1w = ln1w.reshape(_R, 1, h)
    l1b = ln1b.reshape(_R, 1, h)
    l2w = ln2w.reshape(_R, 1, h)
    l2b = ln2b.reshape(_R, 1, h)
    q, k, v = _qkv(sel, Wq, Wk, Wv, l1w, l1b)

    def heads(x):
        return (x.reshape(_R, _B, _K, _NH, _HD)
                 .transpose(0, 1, 3, 2, 4)
                 .reshape(_RB * _NH, _K, _HD))

    ah = _attention(heads(q), heads(k), heads(v))
    attn = (ah.reshape(_R, _B, _NH, _K, _HD)
              .transpose(0, 1, 3, 2, 4)
              .reshape(_R, _M, h))

    h1, hn2 = _oproj(sel, attn, Wo, l2w, l2b)
    w3 = w.reshape(_R, _M, 1)
    weighted = _mlp(hn2, Wg, Wu, Wd, h1, w3)

    wtd = weighted.reshape(_R, _B, _K, 8, 128)
    idx_i32 = idx.astype(jnp.int32)
    total = jnp.stack([
        _scatter_add(hidden_states[bi].reshape(s, 8, 128),
                     wtd[:, bi],
                     idx_i32[:, bi].reshape(_R, 1, _K))
        for bi in range(b)
    ])
    return total.reshape(b, s, h), router_logits


# speedup vs baseline: 1.4662x; 1.4662x over previous
import functools

import jax
import jax.numpy as jnp
from jax import lax
from jax.experimental import pallas as pl
from jax.experimental.pallas import tpu as pltpu
from jax.experimental.pallas import tpu_sc as plsc

_B = 2
_S = 2048
_H = 1024
_NH = 16
_HD = _H // _NH
_I = 4 * _H
_R = 3
_ALPHA = 0.1
_CAP = 0.5
_K = max(1, int(_CAP * _S))
_M = _B * _K
_RB = _R * _B
_ITILE = 256
_IT = _I // _ITILE


def _dotT(a, b, prec=None):
    return lax.dot_general(a, b, (((1,), (1,)), ((), ())),
                           preferred_element_type=jnp.float32,
                           precision=prec)


def _router_body(h_ref, wr_ref, out_ref):
    out_ref[...] = lax.dot_general(
        h_ref[...], wr_ref[...], (((1,), (0,)), ((), ())),
        preferred_element_type=jnp.float32,
        precision=lax.Precision.HIGHEST)


def _router(hidden_flat, wr_pad):
    n = hidden_flat.shape[0]
    tm = 1024
    return pl.pallas_call(
        _router_body,
        grid=(n // tm,),
        in_specs=[
            pl.BlockSpec((tm, _H), lambda i: (i, 0)),
            pl.BlockSpec((_H, 128), lambda i: (0, 0)),
        ],
        out_specs=pl.BlockSpec((tm, 128), lambda i: (i, 0)),
        out_shape=jax.ShapeDtypeStruct((n, 128), jnp.float32),
    )(hidden_flat, wr_pad)


def _sc_gather(table, idx_flat, n_rows):
    info = plsc.get_sparse_core_info()
    nw = info.num_cores * info.num_subcores
    rpw = n_rows // nw
    ch = 64
    mesh = plsc.VectorSubcoreMesh(core_axis_name="c", subcore_axis_name="s")

    @functools.partial(
        pl.kernel, mesh=mesh,
        out_type=jax.ShapeDtypeStruct((n_rows, _H), jnp.float32),
        scratch_types=[
            pltpu.VMEM((ch,), jnp.int32),
            pltpu.VMEM((ch, _H), jnp.float32),
            pltpu.SemaphoreType.DMA,
        ],
    )
    def k(table_hbm, idx_hbm, out_hbm, idx_v, rows_v, sem):
        wid = lax.axis_index("s") * info.num_cores + lax.axis_index("c")
        base = wid * rpw
        for c in range(rpw // ch):
            off = base + c * ch
            pltpu.sync_copy(idx_hbm.at[pl.ds(off, ch)], idx_v)
            pltpu.async_copy(table_hbm.at[idx_v], rows_v, sem).wait()
            pltpu.sync_copy(rows_v, out_hbm.at[pl.ds(off, ch)])

    return k(table, idx_flat)


def _ln(x, w, b):
    mu = jnp.mean(x, axis=-1, keepdims=True)
    xc = x - mu
    var = jnp.mean(xc * xc, axis=-1, keepdims=True)
    return xc * lax.rsqrt(var + 1e-6) * w + b


_MT = 512


def _qkv_body(sel_ref, wq_ref, wk_ref, wv_ref, l1w_ref, l1b_ref,
              q_ref, k_ref, v_ref):
    hn = _ln(sel_ref[...], l1w_ref[...], l1b_ref[...]).astype(jnp.bfloat16)
    q_ref[...] = _dotT(hn, wq_ref[...].astype(jnp.bfloat16)).astype(jnp.bfloat16)
    k_ref[...] = _dotT(hn, wk_ref[...].astype(jnp.bfloat16)).astype(jnp.bfloat16)
    v_ref[...] = _dotT(hn, wv_ref[...].astype(jnp.bfloat16)).astype(jnp.bfloat16)


def _qkv(sel, Wq, Wk, Wv, l1w, l1b):
    spec_w = pl.BlockSpec((None, _H, _H), lambda r, m: (r, 0, 0))
    spec_o = pl.BlockSpec((None, _MT, _H), lambda r, m: (r, m, 0))
    spec_l = pl.BlockSpec((None, 1, _H), lambda r, m: (r, 0, 0))
    return pl.pallas_call(
        _qkv_body,
        grid=(_R, _M // _MT),
        in_specs=[
            pl.BlockSpec((None, _MT, _H), lambda r, m: (r, m, 0)),
            spec_w, spec_w, spec_w, spec_l, spec_l,
        ],
        out_specs=[spec_o, spec_o, spec_o],
        out_shape=[jax.ShapeDtypeStruct((_R, _M, _H), jnp.bfloat16)] * 3,
    )(sel, Wq, Wk, Wv, l1w, l1b)


def _attn_body(q_ref, k_ref, v_ref, o_ref):
    q2 = q_ref[...]
    k2 = k_ref[...]
    v2 = v_ref[...]
    lane = lax.broadcasted_iota(jnp.int32, (1, 2 * _HD), 1)
    scale = 1.0 / (_HD ** 0.5)
    acc = None
    for half in range(2):
        msk = (lane < _HD) if half == 0 else (lane >= _HD)
        qh = jnp.where(msk, q2, jnp.bfloat16(0.0))
        s = lax.dot_general(qh, k2, (((1,), (1,)), ((), ())),
                            preferred_element_type=jnp.float32) * scale
        e = jnp.exp(s)
        p = (e / jnp.sum(e, axis=-1, keepdims=True)).astype(jnp.bfloat16)
        pv = lax.dot_general(p, v2, (((1,), (0,)), ((), ())),
                             preferred_element_type=jnp.float32)
        pv = jnp.where(msk, pv, 0.0)
        acc = pv if acc is None else acc + pv
    o_ref[...] = acc.astype(jnp.bfloat16)


def _attention(q, k, v):
    spec = pl.BlockSpec((None, _K, 2 * _HD), lambda r, b, c: (r, b, c))
    return pl.pallas_call(
        _attn_body,
        grid=(_R, _B, _H // (2 * _HD)),
        in_specs=[spec, spec, spec],
        out_specs=spec,
        out_shape=jax.ShapeDtypeStruct((_R, _M, _H), jnp.bfloat16),
    )(q, k, v)


def _oproj_body(sel_ref, attn_ref, wo_ref, l2w_ref, l2b_ref, h1_ref, hn2_ref):
    h1 = sel_ref[...] + _dotT(attn_ref[...], wo_ref[...].astype(jnp.bfloat16))
    h1_ref[...] = h1.astype(jnp.bfloat16)
    hn2_ref[...] = _ln(h1, l2w_ref[...], l2b_ref[...]).astype(jnp.bfloat16)


def _oproj(sel, attn, Wo, l2w, l2b):
    spec_m = pl.BlockSpec((None, _MT, _H), lambda r, m: (r, m, 0))
    spec_l = pl.BlockSpec((None, 1, _H), lambda r, m: (r, 0, 0))
    return pl.pallas_call(
        _oproj_body,
        grid=(_R, _M // _MT),
        in_specs=[
            spec_m, spec_m,
            pl.BlockSpec((None, _H, _H), lambda r, m: (r, 0, 0)),
            spec_l, spec_l,
        ],
        out_specs=[spec_m, spec_m],
        out_shape=[
            jax.ShapeDtypeStruct((_R, _M, _H), jnp.bfloat16),
            jax.ShapeDtypeStruct((_R, _M, _H), jnp.bfloat16),
        ],
    )(sel, attn, Wo, l2w, l2b)


def _mlp_body(hn2_ref, wg_ref, wu_ref, wd_ref, out_ref):
    hn2 = hn2_ref[...]
    g = _dotT(hn2, wg_ref[...].astype(jnp.bfloat16))
    g = g * jax.nn.sigmoid(g)
    u = _dotT(hn2, wu_ref[...].astype(jnp.bfloat16))
    prod = (g * u).astype(jnp.bfloat16)
    part = _dotT(prod, wd_ref[...].astype(jnp.bfloat16))
    j = pl.program_id(1)

    @pl.when(j == 0)
    def _():
        out_ref[...] = part

    @pl.when(j > 0)
    def _():
        out_ref[...] += part


def _mlp(hn2, Wg, Wu, Wd):
    return pl.pallas_call(
        _mlp_body,
        grid=(_R, _IT),
        in_specs=[
            pl.BlockSpec((None, _M, _H), lambda r, j: (r, 0, 0)),
            pl.BlockSpec((None, _ITILE, _H), lambda r, j: (r, j, 0)),
            pl.BlockSpec((None, _ITILE, _H), lambda r, j: (r, j, 0)),
            pl.BlockSpec((None, _H, _ITILE), lambda r, j: (r, 0, j)),
        ],
        out_specs=pl.BlockSpec((None, _M, _H), lambda r, j: (r, 0, 0)),
        out_shape=jax.ShapeDtypeStruct((_R, _M, _H), jnp.float32),
    )(hn2, Wg, Wu, Wd)


def _scatter_body(hid_ref, h1_ref, mlp_ref, w_ref, idx_ref, out_ref):
    step = pl.program_id(0)

    @pl.when(step == 0)
    def _():
        out_ref[...] = hid_ref[...]

    def body(i, _):
        g = idx_ref[0, i]
        wi = w_ref[0, i]
        upd = (h1_ref[pl.ds(i, 1)].astype(jnp.float32)
               + mlp_ref[pl.ds(i, 1)]) * wi
        out_ref[pl.ds(g, 1)] += upd
        return 0

    lax.fori_loop(0, _K, body, 0)


def _scatter_add(bi, hidden_t, h1_t, mlp_t, w4, idx4):
    return pl.pallas_call(
        _scatter_body,
        grid=(_R,),
        in_specs=[
            pl.BlockSpec((None, _S, 8, 128), lambda i: (bi, 0, 0, 0)),
            pl.BlockSpec((None, None, _K, 8, 128),
                         lambda i: (i, bi, 0, 0, 0)),
            pl.BlockSpec((None, None, _K, 8, 128),
                         lambda i: (i, bi, 0, 0, 0)),
            pl.BlockSpec((None, None, 1, _K), lambda i: (i, bi, 0, 0),
                         memory_space=pltpu.SMEM),
            pl.BlockSpec((None, None, 1, _K), lambda i: (i, bi, 0, 0),
                         memory_space=pltpu.SMEM),
        ],
        out_specs=pl.BlockSpec((_S, 8, 128), lambda i: (0, 0, 0)),
        out_shape=jax.ShapeDtypeStruct((_S, 8, 128), jnp.float32),
    )(hidden_t, h1_t, mlp_t, w4, idx4)


def kernel(hidden_states, Wr, Wq, Wk, Wv, Wo, Wg, Wu, Wd,
           ln1w, ln1b, ln2w, ln2b):
    b, s, h = hidden_states.shape
    hid_flat = hidden_states.reshape(b * s, h)

    wr_pad = jnp.zeros((h, 128), jnp.float32).at[:, :_R].set(
        Wr.reshape(_R, h).T)
    logits_all = _router(hid_flat, wr_pad)
    logits = logits_all.reshape(b, s, 128)[:, :, :_R]
    logits = jnp.transpose(logits, (2, 0, 1))
    router_logits = logits[..., None]

    probs = jax.nn.sigmoid(logits) * _ALPHA
    w, idx = lax.top_k(probs, _K)
    order = jnp.argsort(idx, axis=-1)
    idx = jnp.take_along_axis(idx, order, axis=-1)
    w = jnp.take_along_axis(w, order, axis=-1)
    gidx = idx + jnp.arange(b, dtype=idx.dtype)[None, :, None] * s
    gidx_flat = gidx.reshape(-1).astype(jnp.int32)

    sel = _sc_gather(hid_flat, gidx_flat, _RB * _K)
    sel = sel.reshape(_R, _M, h)

    l1w = ln1w.reshape(_R, 1, h)
    l1b = ln1b.reshape(_R, 1, h)
    l2w = ln2w.reshape(_R, 1, h)
    l2b = ln2b.reshape(_R, 1, h)
    q, k, v = _qkv(sel, Wq, Wk, Wv, l1w, l1b)
    attn = _attention(q, k, v)
    h1, hn2 = _oproj(sel, attn, Wo, l2w, l2b)
    mlp = _mlp(hn2, Wg, Wu, Wd)

    hid4 = hidden_states.reshape(b, s, 8, 128)
    h1_t = h1.reshape(_R, _B, _K, 8, 128)
    mlp_t = mlp.reshape(_R, _B, _K, 8, 128)
    w4 = w.reshape(_R, _B, 1, _K)
    idx4 = idx.reshape(_R, _B, 1, _K).astype(jnp.int32)
    total = jnp.stack([
        _scatter_add(bi, hid4, h1_t, mlp_t, w4, idx4) for bi in range(b)
    ])
    return total.reshape(b, s, h), router_logits


# speedup vs baseline: 1.6304x; 1.1120x over previous
import functools

import jax
import jax.numpy as jnp
from jax import lax
from jax.experimental import pallas as pl
from jax.experimental.pallas import tpu as pltpu
from jax.experimental.pallas import tpu_sc as plsc

_B = 2
_S = 2048
_H = 1024
_NH = 16
_HD = _H // _NH
_I = 4 * _H
_R = 3
_ALPHA = 0.1
_CAP = 0.5
_K = max(1, int(_CAP * _S))
_M = _B * _K
_RB = _R * _B
_ITILE = 512
_IT = _I // _ITILE


def _dotT(a, b, prec=None):
    return lax.dot_general(a, b, (((1,), (1,)), ((), ())),
                           preferred_element_type=jnp.float32,
                           precision=prec)


def _router_body(h_ref, wr_ref, out_ref):
    out_ref[...] = lax.dot_general(
        h_ref[...], wr_ref[...], (((1,), (0,)), ((), ())),
        preferred_element_type=jnp.float32,
        precision=lax.Precision.HIGHEST)


def _router(hidden_flat, wr_pad):
    n = hidden_flat.shape[0]
    tm = 1024
    return pl.pallas_call(
        _router_body,
        grid=(n // tm,),
        in_specs=[
            pl.BlockSpec((tm, _H), lambda i: (i, 0)),
            pl.BlockSpec((_H, 128), lambda i: (0, 0)),
        ],
        out_specs=pl.BlockSpec((tm, 128), lambda i: (i, 0)),
        out_shape=jax.ShapeDtypeStruct((n, 128), jnp.float32),
    )(hidden_flat, wr_pad)


def _sc_gather(table, idx_flat, n_rows):
    info = plsc.get_sparse_core_info()
    nw = info.num_cores * info.num_subcores
    rpw = n_rows // nw
    ch = 64
    mesh = plsc.VectorSubcoreMesh(core_axis_name="c", subcore_axis_name="s")

    @functools.partial(
        pl.kernel, mesh=mesh,
        out_type=jax.ShapeDtypeStruct((n_rows, _H), jnp.float32),
        scratch_types=[
            pltpu.VMEM((ch,), jnp.int32),
            pltpu.VMEM((ch, _H), jnp.float32),
            pltpu.SemaphoreType.DMA,
        ],
    )
    def k(table_hbm, idx_hbm, out_hbm, idx_v, rows_v, sem):
        wid = lax.axis_index("s") * info.num_cores + lax.axis_index("c")
        base = wid * rpw
        for c in range(rpw // ch):
            off = base + c * ch
            pltpu.sync_copy(idx_hbm.at[pl.ds(off, ch)], idx_v)
            pltpu.async_copy(table_hbm.at[idx_v], rows_v, sem).wait()
            pltpu.sync_copy(rows_v, out_hbm.at[pl.ds(off, ch)])

    return k(table, idx_flat)


def _ln(x, w, b):
    mu = jnp.mean(x, axis=-1, keepdims=True)
    xc = x - mu
    var = jnp.mean(xc * xc, axis=-1, keepdims=True)
    return xc * lax.rsqrt(var + 1e-6) * w + b


_MT = 512


def _qkv_body(sel_ref, wq_ref, wk_ref, wv_ref, l1w_ref, l1b_ref,
              q_ref, k_ref, v_ref):
    hn = _ln(sel_ref[...], l1w_ref[...], l1b_ref[...]).astype(jnp.bfloat16)
    q_ref[...] = _dotT(hn, wq_ref[...].astype(jnp.bfloat16)).astype(jnp.bfloat16)
    k_ref[...] = _dotT(hn, wk_ref[...].astype(jnp.bfloat16)).astype(jnp.bfloat16)
    v_ref[...] = _dotT(hn, wv_ref[...].astype(jnp.bfloat16)).astype(jnp.bfloat16)


def _qkv(sel, Wq, Wk, Wv, l1w, l1b):
    spec_w = pl.BlockSpec((None, _H, _H), lambda r, m: (r, 0, 0))
    spec_o = pl.BlockSpec((None, _MT, _H), lambda r, m: (r, m, 0))
    spec_l = pl.BlockSpec((None, 1, _H), lambda r, m: (r, 0, 0))
    return pl.pallas_call(
        _qkv_body,
        grid=(_R, _M // _MT),
        in_specs=[
            pl.BlockSpec((None, _MT, _H), lambda r, m: (r, m, 0)),
            spec_w, spec_w, spec_w, spec_l, spec_l,
        ],
        out_specs=[spec_o, spec_o, spec_o],
        out_shape=[jax.ShapeDtypeStruct((_R, _M, _H), jnp.bfloat16)] * 3,
    )(sel, Wq, Wk, Wv, l1w, l1b)


def _attn_body(q_ref, k_ref, v_ref, o_ref):
    q2 = q_ref[...]
    k2 = k_ref[...]
    v2 = v_ref[...]
    lane = lax.broadcasted_iota(jnp.int32, (1, 2 * _HD), 1)
    scale = 1.0 / (_HD ** 0.5)
    acc = None
    for half in range(2):
        msk = (lane < _HD) if half == 0 else (lane >= _HD)
        qh = jnp.where(msk, q2, jnp.bfloat16(0.0))
        s = lax.dot_general(qh, k2, (((1,), (1,)), ((), ())),
                            preferred_element_type=jnp.float32) * scale
        e = jnp.exp(s)
        inv = 1.0 / jnp.sum(e, axis=-1, keepdims=True)
        pv = lax.dot_general(e.astype(jnp.bfloat16), v2,
                             (((1,), (0,)), ((), ())),
                             preferred_element_type=jnp.float32)
        pv = jnp.where(msk, pv * inv, 0.0)
        acc = pv if acc is None else acc + pv
    o_ref[...] = acc.astype(jnp.bfloat16)


def _attention(q, k, v):
    spec = pl.BlockSpec((None, _K, 2 * _HD), lambda r, b, c: (r, b, c))
    return pl.pallas_call(
        _attn_body,
        grid=(_R, _B, _H // (2 * _HD)),
        in_specs=[spec, spec, spec],
        out_specs=spec,
        out_shape=jax.ShapeDtypeStruct((_R, _M, _H), jnp.bfloat16),
    )(q, k, v)


def _oproj_body(sel_ref, attn_ref, wo_ref, l2w_ref, l2b_ref, w_ref,
                h1w_ref, hn2_ref):
    h1 = sel_ref[...] + _dotT(attn_ref[...], wo_ref[...].astype(jnp.bfloat16))
    h1w_ref[...] = (h1 * w_ref[...]).astype(jnp.bfloat16)
    hn2_ref[...] = _ln(h1, l2w_ref[...], l2b_ref[...]).astype(jnp.bfloat16)


def _oproj(sel, attn, Wo, l2w, l2b, w3):
    spec_m = pl.BlockSpec((None, _MT, _H), lambda r, m: (r, m, 0))
    spec_l = pl.BlockSpec((None, 1, _H), lambda r, m: (r, 0, 0))
    return pl.pallas_call(
        _oproj_body,
        grid=(_R, _M // _MT),
        in_specs=[
            spec_m, spec_m,
            pl.BlockSpec((None, _H, _H), lambda r, m: (r, 0, 0)),
            spec_l, spec_l,
            pl.BlockSpec((None, _MT, 1), lambda r, m: (r, m, 0)),
        ],
        out_specs=[spec_m, spec_m],
        out_shape=[
            jax.ShapeDtypeStruct((_R, _M, _H), jnp.bfloat16),
            jax.ShapeDtypeStruct((_R, _M, _H), jnp.bfloat16),
        ],
    )(sel, attn, Wo, l2w, l2b, w3)


def _mlp_body(hn2_ref, wg_ref, wu_ref, wd_ref, w_ref, out_ref):
    hn2 = hn2_ref[...]
    g = _dotT(hn2, wg_ref[...].astype(jnp.bfloat16))
    g = g * jax.nn.sigmoid(g)
    u = _dotT(hn2, wu_ref[...].astype(jnp.bfloat16))
    prod = (g * u).astype(jnp.bfloat16)
    part = _dotT(prod, wd_ref[...].astype(jnp.bfloat16))
    j = pl.program_id(1)

    @pl.when(j == 0)
    def _():
        out_ref[...] = part.astype(jnp.bfloat16)

    @pl.when((j > 0) & (j < _IT - 1))
    def _():
        out_ref[...] = (out_ref[...].astype(jnp.float32)
                        + part).astype(jnp.bfloat16)

    @pl.when(j == _IT - 1)
    def _():
        out_ref[...] = ((out_ref[...].astype(jnp.float32) + part)
                        * w_ref[...]).astype(jnp.bfloat16)


def _mlp(hn2, Wg, Wu, Wd, w3):
    return pl.pallas_call(
        _mlp_body,
        grid=(_R, _IT),
        in_specs=[
            pl.BlockSpec((None, _M, _H), lambda r, j: (r, 0, 0)),
            pl.BlockSpec((None, _ITILE, _H), lambda r, j: (r, j, 0)),
            pl.BlockSpec((None, _ITILE, _H), lambda r, j: (r, j, 0)),
            pl.BlockSpec((None, _H, _ITILE), lambda r, j: (r, 0, j)),
            pl.BlockSpec((None, _M, 1), lambda r, j: (r, 0, 0)),
        ],
        out_specs=pl.BlockSpec((None, _M, _H), lambda r, j: (r, 0, 0)),
        out_shape=jax.ShapeDtypeStruct((_R, _M, _H), jnp.bfloat16),
    )(hn2, Wg, Wu, Wd, w3)


def _scatter_body(hid_ref, h1w_ref, mlpw_ref, idx_ref, out_ref):
    step = pl.program_id(0)

    @pl.when(step == 0)
    def _():
        out_ref[...] = hid_ref[...]

    def body(i, _):
        g = idx_ref[0, i]
        upd = (h1w_ref[pl.ds(i, 1)].astype(jnp.float32)
               + mlpw_ref[pl.ds(i, 1)].astype(jnp.float32))
        out_ref[pl.ds(g, 1)] += upd
        return 0

    lax.fori_loop(0, _K, body, 0)


def _scatter_add(bi, hidden_t, h1w_t, mlpw_t, idx4):
    return pl.pallas_call(
        _scatter_body,
        grid=(_R,),
        in_specs=[
            pl.BlockSpec((None, _S, 8, 128), lambda i: (bi, 0, 0, 0)),
            pl.BlockSpec((None, None, _K, 8, 128),
                         lambda i: (i, bi, 0, 0, 0)),
            pl.BlockSpec((None, None, _K, 8, 128),
                         lambda i: (i, bi, 0, 0, 0)),
            pl.BlockSpec((None, None, 1, _K), lambda i: (i, bi, 0, 0),
                         memory_space=pltpu.SMEM),
        ],
        out_specs=pl.BlockSpec((_S, 8, 128), lambda i: (0, 0, 0)),
        out_shape=jax.ShapeDtypeStruct((_S, 8, 128), jnp.float32),
    )(hidden_t, h1w_t, mlpw_t, idx4)


def kernel(hidden_states, Wr, Wq, Wk, Wv, Wo, Wg, Wu, Wd,
           ln1w, ln1b, ln2w, ln2b):
    b, s, h = hidden_states.shape
    hid_flat = hidden_states.reshape(b * s, h)

    wr_pad = jnp.zeros((h, 128), jnp.float32).at[:, :_R].set(
        Wr.reshape(_R, h).T)
    logits_all = _router(hid_flat, wr_pad)
    logits = logits_all.reshape(b, s, 128)[:, :, :_R]
    logits = jnp.transpose(logits, (2, 0, 1))
    router_logits = logits[..., None]

    probs = jax.nn.sigmoid(logits) * _ALPHA
    w, idx = lax.top_k(probs, _K)
    order = jnp.argsort(idx, axis=-1)
    idx = jnp.take_along_axis(idx, order, axis=-1)
    w = jnp.take_along_axis(w, order, axis=-1)
    gidx = idx + jnp.arange(b, dtype=idx.dtype)[None, :, None] * s
    gidx_flat = gidx.reshape(-1).astype(jnp.int32)

    sel = _sc_gather(hid_flat, gidx_flat, _RB * _K)
    sel = sel.reshape(_R, _M, h)

    l1w = ln1w.reshape(_R, 1, h)
    l1b = ln1b.reshape(_R, 1, h)
    l2w = ln2w.reshape(_R, 1, h)
    l2b = ln2b.reshape(_R, 1, h)
    w3 = w.reshape(_R, _M, 1)
    q, k, v = _qkv(sel, Wq, Wk, Wv, l1w, l1b)
    attn = _attention(q, k, v)
    h1w, hn2 = _oproj(sel, attn, Wo, l2w, l2b, w3)
    mlpw = _mlp(hn2, Wg, Wu, Wd, w3)

    hid4 = hidden_states.reshape(b, s, 8, 128)
    h1w_t = h1w.reshape(_R, _B, _K, 8, 128)
    mlpw_t = mlpw.reshape(_R, _B, _K, 8, 128)
    idx4 = idx.reshape(_R, _B, 1, _K).astype(jnp.int32)
    total = jnp.stack([
        _scatter_add(bi, hid4, h1w_t, mlpw_t, idx4) for bi in range(b)
    ])
    return total.reshape(b, s, h), router_logits


# speedup vs baseline: 1.7145x; 1.0516x over previous
import functools

import jax
import jax.numpy as jnp
from jax import lax
from jax.experimental import pallas as pl
from jax.experimental.pallas import tpu as pltpu
from jax.experimental.pallas import tpu_sc as plsc

_B = 2
_S = 2048
_H = 1024
_NH = 16
_HD = _H // _NH
_I = 4 * _H
_R = 3
_ALPHA = 0.1
_CAP = 0.5
_K = max(1, int(_CAP * _S))
_M = _B * _K
_RB = _R * _B
_ITILE = 512
_IT = _I // _ITILE


def _dotT(a, b, prec=None):
    return lax.dot_general(a, b, (((1,), (1,)), ((), ())),
                           preferred_element_type=jnp.float32,
                           precision=prec)


def _router_body(h_ref, wr_ref, out_ref):
    out_ref[...] = lax.dot_general(
        h_ref[...], wr_ref[...], (((1,), (0,)), ((), ())),
        preferred_element_type=jnp.float32,
        precision=lax.Precision.HIGHEST)


def _router(hidden_flat, wr_pad):
    n = hidden_flat.shape[0]
    tm = 1024
    return pl.pallas_call(
        _router_body,
        grid=(n // tm,),
        in_specs=[
            pl.BlockSpec((tm, _H), lambda i: (i, 0)),
            pl.BlockSpec((_H, 128), lambda i: (0, 0)),
        ],
        out_specs=pl.BlockSpec((tm, 128), lambda i: (i, 0)),
        out_shape=jax.ShapeDtypeStruct((n, 128), jnp.float32),
    )(hidden_flat, wr_pad)


def _sc_gather(table, idx_flat, n_rows):
    info = plsc.get_sparse_core_info()
    nw = info.num_cores * info.num_subcores
    rpw = n_rows // nw
    ch = 64
    mesh = plsc.VectorSubcoreMesh(core_axis_name="c", subcore_axis_name="s")

    @functools.partial(
        pl.kernel, mesh=mesh,
        out_type=jax.ShapeDtypeStruct((n_rows, _H), jnp.float32),
        scratch_types=[
            pltpu.VMEM((ch,), jnp.int32),
            pltpu.VMEM((ch, _H), jnp.float32),
            pltpu.SemaphoreType.DMA,
        ],
    )
    def k(table_hbm, idx_hbm, out_hbm, idx_v, rows_v, sem):
        wid = lax.axis_index("s") * info.num_cores + lax.axis_index("c")
        base = wid * rpw
        for c in range(rpw // ch):
            off = base + c * ch
            pltpu.sync_copy(idx_hbm.at[pl.ds(off, ch)], idx_v)
            pltpu.async_copy(table_hbm.at[idx_v], rows_v, sem).wait()
            pltpu.sync_copy(rows_v, out_hbm.at[pl.ds(off, ch)])

    return k(table, idx_flat)


def _ln(x, w, b):
    mu = jnp.mean(x, axis=-1, keepdims=True)
    xc = x - mu
    var = jnp.mean(xc * xc, axis=-1, keepdims=True)
    return xc * lax.rsqrt(var + 1e-6) * w + b


_MT = 512


def _qkv_body(sel_ref, wq_ref, wk_ref, wv_ref, l1w_ref, l1b_ref,
              q_ref, k_ref, v_ref):
    hn = _ln(sel_ref[...], l1w_ref[...], l1b_ref[...]).astype(jnp.bfloat16)
    q_ref[...] = _dotT(hn, wq_ref[...].astype(jnp.bfloat16)).astype(jnp.bfloat16)
    k_ref[...] = _dotT(hn, wk_ref[...].astype(jnp.bfloat16)).astype(jnp.bfloat16)
    v_ref[...] = _dotT(hn, wv_ref[...].astype(jnp.bfloat16)).astype(jnp.bfloat16)


def _qkv(sel, Wq, Wk, Wv, l1w, l1b):
    spec_w = pl.BlockSpec((None, _H, _H), lambda r, m: (r, 0, 0))
    spec_o = pl.BlockSpec((None, _MT, _H), lambda r, m: (r, m, 0))
    spec_l = pl.BlockSpec((None, 1, _H), lambda r, m: (r, 0, 0))
    return pl.pallas_call(
        _qkv_body,
        grid=(_R, _M // _MT),
        in_specs=[
            pl.BlockSpec((None, _MT, _H), lambda r, m: (r, m, 0)),
            spec_w, spec_w, spec_w, spec_l, spec_l,
        ],
        out_specs=[spec_o, spec_o, spec_o],
        out_shape=[jax.ShapeDtypeStruct((_R, _M, _H), jnp.bfloat16)] * 3,
    )(sel, Wq, Wk, Wv, l1w, l1b)


def _attn_body(q_ref, k_ref, v_ref, o_ref):
    q2 = q_ref[...]
    k2 = k_ref[...]
    v2 = v_ref[...]
    lane = lax.broadcasted_iota(jnp.int32, (1, 2 * _HD), 1)
    scale = 1.0 / (_HD ** 0.5)
    acc = None
    for half in range(2):
        msk = (lane < _HD) if half == 0 else (lane >= _HD)
        qh = jnp.where(msk, q2, jnp.bfloat16(0.0))
        s = lax.dot_general(qh, k2, (((1,), (1,)), ((), ())),
                            preferred_element_type=jnp.float32) * scale
        e = jnp.exp(s)
        inv = 1.0 / jnp.sum(e, axis=-1, keepdims=True)
        pv = lax.dot_general(e.astype(jnp.bfloat16), v2,
                             (((1,), (0,)), ((), ())),
                             preferred_element_type=jnp.float32)
        pv = jnp.where(msk, pv * inv, 0.0)
        acc = pv if acc is None else acc + pv
    o_ref[...] = acc.astype(jnp.bfloat16)


def _attention(q, k, v):
    spec = pl.BlockSpec((None, _K, 2 * _HD), lambda r, b, c: (r, b, c))
    return pl.pallas_call(
        _attn_body,
        grid=(_R, _B, _H // (2 * _HD)),
        in_specs=[spec, spec, spec],
        out_specs=spec,
        out_shape=jax.ShapeDtypeStruct((_R, _M, _H), jnp.bfloat16),
    )(q, k, v)


def _oproj_body(sel_ref, attn_ref, wo_ref, l2w_ref, l2b_ref, w_ref,
                h1w_ref, hn2_ref):
    h1 = sel_ref[...] + _dotT(attn_ref[...], wo_ref[...].astype(jnp.bfloat16))
    h1w_ref[...] = (h1 * w_ref[...]).astype(jnp.bfloat16)
    hn2_ref[...] = _ln(h1, l2w_ref[...], l2b_ref[...]).astype(jnp.bfloat16)


def _oproj(sel, attn, Wo, l2w, l2b, w3):
    spec_m = pl.BlockSpec((None, _MT, _H), lambda r, m: (r, m, 0))
    spec_l = pl.BlockSpec((None, 1, _H), lambda r, m: (r, 0, 0))
    return pl.pallas_call(
        _oproj_body,
        grid=(_R, _M // _MT),
        in_specs=[
            spec_m, spec_m,
            pl.BlockSpec((None, _H, _H), lambda r, m: (r, 0, 0)),
            spec_l, spec_l,
            pl.BlockSpec((None, _MT, 1), lambda r, m: (r, m, 0)),
        ],
        out_specs=[spec_m, spec_m],
        out_shape=[
            jax.ShapeDtypeStruct((_R, _M, _H), jnp.bfloat16),
            jax.ShapeDtypeStruct((_R, _M, _H), jnp.bfloat16),
        ],
    )(sel, attn, Wo, l2w, l2b, w3)


def _mlp_body(hn2_ref, wg_ref, wu_ref, wd_ref, w_ref, out_ref):
    hn2 = hn2_ref[...]
    g = _dotT(hn2, wg_ref[...].astype(jnp.bfloat16))
    g = g * jax.nn.sigmoid(g)
    u = _dotT(hn2, wu_ref[...].astype(jnp.bfloat16))
    prod = (g * u).astype(jnp.bfloat16)
    part = _dotT(prod, wd_ref[...].astype(jnp.bfloat16))
    j = pl.program_id(1)

    @pl.when(j == 0)
    def _():
        out_ref[...] = part.astype(jnp.bfloat16)

    @pl.when((j > 0) & (j < _IT - 1))
    def _():
        out_ref[...] = (out_ref[...].astype(jnp.float32)
                        + part).astype(jnp.bfloat16)

    @pl.when(j == _IT - 1)
    def _():
        out_ref[...] = ((out_ref[...].astype(jnp.float32) + part)
                        * w_ref[...]).astype(jnp.bfloat16)


def _mlp(hn2, Wg, Wu, Wd, w3):
    return pl.pallas_call(
        _mlp_body,
        grid=(_R, _IT),
        in_specs=[
            pl.BlockSpec((None, _M, _H), lambda r, j: (r, 0, 0)),
            pl.BlockSpec((None, _ITILE, _H), lambda r, j: (r, j, 0)),
            pl.BlockSpec((None, _ITILE, _H), lambda r, j: (r, j, 0)),
            pl.BlockSpec((None, _H, _ITILE), lambda r, j: (r, 0, j)),
            pl.BlockSpec((None, _M, 1), lambda r, j: (r, 0, 0)),
        ],
        out_specs=pl.BlockSpec((None, _M, _H), lambda r, j: (r, 0, 0)),
        out_shape=jax.ShapeDtypeStruct((_R, _M, _H), jnp.bfloat16),
    )(hn2, Wg, Wu, Wd, w3)


def _scatter_body(hid_ref, h1w_ref, mlpw_ref, idx_ref, out_ref):
    step = pl.program_id(0)

    @pl.when(step == 0)
    def _():
        out_ref[...] = hid_ref[...]

    def body(i4, _):
        base = i4 * 4
        upd = (h1w_ref[pl.ds(base, 4)].astype(jnp.float32)
               + mlpw_ref[pl.ds(base, 4)].astype(jnp.float32))
        for j in range(4):
            g = idx_ref[0, base + j]
            out_ref[pl.ds(g, 1)] += upd[j:j + 1]
        return 0

    lax.fori_loop(0, _K // 4, body, 0)


def _scatter_add(bi, hidden_t, h1w_t, mlpw_t, idx4):
    return pl.pallas_call(
        _scatter_body,
        grid=(_R,),
        in_specs=[
            pl.BlockSpec((None, _S, 8, 128), lambda i: (bi, 0, 0, 0)),
            pl.BlockSpec((None, None, _K, 8, 128),
                         lambda i: (i, bi, 0, 0, 0)),
            pl.BlockSpec((None, None, _K, 8, 128),
                         lambda i: (i, bi, 0, 0, 0)),
            pl.BlockSpec((None, None, 1, _K), lambda i: (i, bi, 0, 0),
                         memory_space=pltpu.SMEM),
        ],
        out_specs=pl.BlockSpec((_S, 8, 128), lambda i: (0, 0, 0)),
        out_shape=jax.ShapeDtypeStruct((_S, 8, 128), jnp.float32),
    )(hidden_t, h1w_t, mlpw_t, idx4)


def kernel(hidden_states, Wr, Wq, Wk, Wv, Wo, Wg, Wu, Wd,
           ln1w, ln1b, ln2w, ln2b):
    b, s, h = hidden_states.shape
    hid_flat = hidden_states.reshape(b * s, h)

    wr_pad = jnp.zeros((h, 128), jnp.float32).at[:, :_R].set(
        Wr.reshape(_R, h).T)
    logits_all = _router(hid_flat, wr_pad)
    logits = logits_all.reshape(b, s, 128)[:, :, :_R]
    logits = jnp.transpose(logits, (2, 0, 1))
    router_logits = logits[..., None]

    probs = jax.nn.sigmoid(logits) * _ALPHA
    w, idx = lax.top_k(probs, _K)
    idx, w = lax.sort((idx, w), dimension=-1, num_keys=1)
    gidx = idx + jnp.arange(b, dtype=idx.dtype)[None, :, None] * s
    gidx_flat = gidx.reshape(-1).astype(jnp.int32)

    sel = _sc_gather(hid_flat, gidx_flat, _RB * _K)
    sel = sel.reshape(_R, _M, h)

    l1w = ln1w.reshape(_R, 1, h)
    l1b = ln1b.reshape(_R, 1, h)
    l2w = ln2w.reshape(_R, 1, h)
    l2b = ln2b.reshape(_R, 1, h)
    w3 = w.reshape(_R, _M, 1)
    q, k, v = _qkv(sel, Wq, Wk, Wv, l1w, l1b)
    attn = _attention(q, k, v)
    h1w, hn2 = _oproj(sel, attn, Wo, l2w, l2b, w3)
    mlpw = _mlp(hn2, Wg, Wu, Wd, w3)

    hid4 = hidden_states.reshape(b, s, 8, 128)
    h1w_t = h1w.reshape(_R, _B, _K, 8, 128)
    mlpw_t = mlpw.reshape(_R, _B, _K, 8, 128)
    idx4 = idx.reshape(_R, _B, 1, _K).astype(jnp.int32)
    total = jnp.stack([
        _scatter_add(bi, hid4, h1w_t, mlpw_t, idx4) for bi in range(b)
    ])
    return total.reshape(b, s, h), router_logits
